# baseline (speedup 1.0000x reference)
import functools

import jax
import jax.numpy as jnp
from jax import lax
from jax.experimental import pallas as pl
from jax.experimental.pallas import tpu as pltpu
from jax.experimental.pallas import tpu_sc as plsc

LANES = 16
NUM_TILES = 16
NUM_IDX = 16384
PER_TILE = NUM_IDX // NUM_TILES
VECS_PER_TILE = PER_TILE // LANES

_mesh = plsc.VectorSubcoreMesh(
    core_axis_name="c", subcore_axis_name="s", num_cores=1, num_subcores=16
)

_GATHER_DNUMS = lax.GatherDimensionNumbers(
    offset_dims=(), collapsed_slice_dims=(0,), start_index_map=(0,))


def _perm(v, idx):
    return lax.gather(v, idx[:, None], _GATHER_DNUMS, slice_sizes=(1,),
                      mode=lax.GatherScatterMode.PROMISE_IN_BOUNDS)


def _lane_allsum(v):
    io = lax.iota(jnp.int32, LANES)
    for sh in (8, 4, 2, 1):
        v = v + _perm(v, lax.bitwise_and(io + sh, LANES - 1))
    return v


def _body(d_hbm, idx_hbm, em_hbm, wb_hbm, wt_hbm, out_hbm,
          part_hbm, idx_v, prm_v, stage_v, all_v, out_v):
    sid = lax.axis_index("s")
    base = sid * PER_TILE

    pltpu.sync_copy(idx_hbm.at[pl.ds(base, PER_TILE)], idx_v)

    UNROLL = 4

    def step(i, carry):
        s1, s2 = carry
        for u in range(UNROLL):
            v = idx_v[pl.ds((i * UNROLL + u) * LANES, LANES)]
            s1 = s1 + v
            s2 = s2 + v * v
        return s1, s2

    zero = jnp.zeros((LANES,), jnp.int32)
    s1, s2 = lax.fori_loop(0, VECS_PER_TILE // UNROLL, step, (zero, zero))

    pltpu.sync_copy(d_hbm, prm_v.at[0, pl.ds(0, 2)])
    pltpu.sync_copy(em_hbm, prm_v.at[1, pl.ds(0, 6)])
    pltpu.sync_copy(wb_hbm, prm_v.at[2, pl.ds(0, 4)])
    pltpu.sync_copy(wt_hbm, prm_v.at[3, pl.ds(0, 4)])

    io = lax.iota(jnp.int32, LANES)
    zerof = jnp.zeros((LANES,), jnp.float32)
    m15 = LANES - 1
    emv = prm_v[1, :]
    wtv = prm_v[3, :]
    q = jnp.where(
        io < 6,
        _perm(emv, lax.bitwise_and(io, m15)) *
        _perm(wtv, 2 + lax.bitwise_and(io, 1)),
        zerof)
    sc2n = 2.0 / NUM_IDX
    mbeta = jnp.where(io < 2, -1.5 * sc2n,
                      jnp.where(io < 4, 2.0 * sc2n,
                                jnp.where(io < 6, -0.5 * sc2n, 0.0)))
    mgamma = jnp.where(io < 2, 0.5 * sc2n,
                       jnp.where(io < 4, -1.0 * sc2n,
                                 jnp.where(io < 6, 0.5 * sc2n, 0.0)))
    beta = _lane_allsum(q * mbeta)
    gamma = _lane_allsum(q * mgamma)
    stage_v[0, :] = (beta * s1.astype(jnp.float32) +
                     gamma * s2.astype(jnp.float32))
    pltpu.sync_copy(stage_v, part_hbm.at[pl.ds(sid, 1)])
    plsc.subcore_barrier()

    @pl.when(sid == 0)
    def _finish():
        pltpu.sync_copy(part_hbm, all_v)
        acc = all_v[0, :]
        for t in range(1, NUM_TILES):
            acc = acc + all_v[t, :]
        dv = prm_v[0, :]
        wbv = prm_v[2, :]
        lt4 = io < 4
        dense_terms = jnp.where(
            lt4,
            _perm(dv, lax.bitwise_and(io, 1)) * wbv *
            _perm(wtv, lax.bitwise_and(lax.shift_right_logical(io, 1), 1)),
            zerof)
        const_terms = jnp.where(io < 2, 2.0 * q, zerof)
        out_v[...] = _lane_allsum(acc + dense_terms + const_terms)
        pltpu.sync_copy(out_v.at[pl.ds(0, 1)], out_hbm)


_sc_call = functools.partial(
    pl.kernel,
    out_type=jax.ShapeDtypeStruct((1,), jnp.float32),
    mesh=_mesh,
    scratch_types=[
        pltpu.HBM((NUM_TILES, LANES), jnp.float32),
        pltpu.VMEM((PER_TILE,), jnp.int32),
        pltpu.VMEM((4, LANES), jnp.float32),
        pltpu.VMEM((1, LANES), jnp.float32),
        pltpu.VMEM((NUM_TILES, LANES), jnp.float32),
        pltpu.VMEM((LANES,), jnp.float32),
    ],
)(_body)


def kernel(dense_features, sparse_features, em_weight, W_bot, W_top):
    out1 = _sc_call(
        dense_features.reshape(-1),
        sparse_features,
        em_weight.reshape(-1),
        W_bot.reshape(-1),
        W_top.reshape(-1),
    )
    return out1.reshape(1, 1)


# speedup vs baseline: 1.0891x; 1.0891x over previous
import functools

import jax
import jax.numpy as jnp
from jax import lax
from jax.experimental import pallas as pl
from jax.experimental.pallas import tpu as pltpu
from jax.experimental.pallas import tpu_sc as plsc

LANES = 16
NUM_TILES = 16
NUM_IDX = 16384
PER_TILE = NUM_IDX // NUM_TILES
VECS_PER_TILE = PER_TILE // LANES

_mesh = plsc.VectorSubcoreMesh(
    core_axis_name="c", subcore_axis_name="s", num_cores=1, num_subcores=16
)

_GATHER_DNUMS = lax.GatherDimensionNumbers(
    offset_dims=(), collapsed_slice_dims=(0,), start_index_map=(0,))


def _perm(v, idx):
    return lax.gather(v, idx[:, None], _GATHER_DNUMS, slice_sizes=(1,),
                      mode=lax.GatherScatterMode.PROMISE_IN_BOUNDS)


def _lane_allsum(v):
    io = lax.iota(jnp.int32, LANES)
    for sh in (8, 4, 2, 1):
        v = v + _perm(v, lax.bitwise_and(io + sh, LANES - 1))
    return v


def _body(d_hbm, idx_hbm, em_hbm, wb_hbm, wt_hbm, out_hbm,
          part_hbm, idx_v, prm_v, stage_v, all_v, out_v, psem):
    sid = lax.axis_index("s")
    base = sid * PER_TILE

    hs = [pltpu.async_copy(d_hbm, prm_v.at[0, pl.ds(0, 2)], psem),
          pltpu.async_copy(em_hbm, prm_v.at[1, pl.ds(0, 6)], psem),
          pltpu.async_copy(wb_hbm, prm_v.at[2, pl.ds(0, 4)], psem),
          pltpu.async_copy(wt_hbm, prm_v.at[3, pl.ds(0, 4)], psem)]

    pltpu.sync_copy(idx_hbm.at[pl.ds(base, PER_TILE)], idx_v)

    UNROLL = 4

    def step(i, carry):
        s1, s2 = carry
        for u in range(UNROLL):
            v = idx_v[pl.ds((i * UNROLL + u) * LANES, LANES)]
            s1 = s1 + v
            s2 = s2 + v * v
        return s1, s2

    zero = jnp.zeros((LANES,), jnp.int32)
    s1, s2 = lax.fori_loop(0, VECS_PER_TILE // UNROLL, step, (zero, zero))

    for h in hs:
        h.wait()

    io = lax.iota(jnp.int32, LANES)
    zerof = jnp.zeros((LANES,), jnp.float32)
    m15 = LANES - 1
    emv = prm_v[1, :]
    wtv = prm_v[3, :]
    q = jnp.where(
        io < 6,
        _perm(emv, lax.bitwise_and(io, m15)) *
        _perm(wtv, 2 + lax.bitwise_and(io, 1)),
        zerof)
    sc2n = 2.0 / NUM_IDX
    mbeta = jnp.where(io < 2, -1.5 * sc2n,
                      jnp.where(io < 4, 2.0 * sc2n,
                                jnp.where(io < 6, -0.5 * sc2n, 0.0)))
    mgamma = jnp.where(io < 2, 0.5 * sc2n,
                       jnp.where(io < 4, -1.0 * sc2n,
                                 jnp.where(io < 6, 0.5 * sc2n, 0.0)))
    beta = _lane_allsum(q * mbeta)
    gamma = _lane_allsum(q * mgamma)
    stage_v[0, :] = (beta * s1.astype(jnp.float32) +
                     gamma * s2.astype(jnp.float32))
    pltpu.sync_copy(stage_v, part_hbm.at[pl.ds(sid, 1)])
    plsc.subcore_barrier()

    @pl.when(sid == 0)
    def _finish():
        pltpu.sync_copy(part_hbm, all_v)
        acc = all_v[0, :]
        for t in range(1, NUM_TILES):
            acc = acc + all_v[t, :]
        dv = prm_v[0, :]
        wbv = prm_v[2, :]
        lt4 = io < 4
        dense_terms = jnp.where(
            lt4,
            _perm(dv, lax.bitwise_and(io, 1)) * wbv *
            _perm(wtv, lax.bitwise_and(lax.shift_right_logical(io, 1), 1)),
            zerof)
        const_terms = jnp.where(io < 2, 2.0 * q, zerof)
        out_v[...] = _lane_allsum(acc + dense_terms + const_terms)
        pltpu.sync_copy(out_v.at[pl.ds(0, 1)], out_hbm)


_sc_call = functools.partial(
    pl.kernel,
    out_type=jax.ShapeDtypeStruct((1,), jnp.float32),
    mesh=_mesh,
    scratch_types=[
        pltpu.HBM((NUM_TILES, LANES), jnp.float32),
        pltpu.VMEM((PER_TILE,), jnp.int32),
        pltpu.VMEM((4, LANES), jnp.float32),
        pltpu.VMEM((1, LANES), jnp.float32),
        pltpu.VMEM((NUM_TILES, LANES), jnp.float32),
        pltpu.VMEM((LANES,), jnp.float32),
        pltpu.SemaphoreType.DMA,
    ],
)(_body)


def kernel(dense_features, sparse_features, em_weight, W_bot, W_top):
    out1 = _sc_call(
        dense_features.reshape(-1),
        sparse_features,
        em_weight.reshape(-1),
        W_bot.reshape(-1),
        W_top.reshape(-1),
    )
    return out1.reshape(1, 1)
